# baseline (speedup 1.0000x reference)
import functools

import jax
import jax.numpy as jnp
from jax import lax
from jax.experimental import pallas as pl
from jax.experimental.pallas import tpu as pltpu
from jax.experimental.pallas import tpu_sc as plsc

B = 16384
D = 64
H = 128

NC = 2
NS = 16
NW = NC * NS
BPW = B // NW
CH = 128
NCH = BPW // CH


def _sc_gather(user2d, movie2d, user_table, movie_table):
  mesh = plsc.VectorSubcoreMesh(core_axis_name="c", subcore_axis_name="s")

  @functools.partial(
      pl.kernel,
      mesh=mesh,
      out_type=[
          jax.ShapeDtypeStruct((B, D), jnp.float32),
          jax.ShapeDtypeStruct((B, D), jnp.float32),
      ],
      scratch_types=[
          pltpu.VMEM((NCH, CH), jnp.int32),
          pltpu.VMEM((NCH, CH), jnp.int32),
          pltpu.VMEM((BPW, D), jnp.float32),
          pltpu.VMEM((BPW, D), jnp.float32),
          pltpu.SemaphoreType.DMA,
      ],
      compiler_params=pltpu.CompilerParams(use_tc_tiling_on_sc=False),
  )
  def k(user_hbm, movie_hbm, ut_hbm, mt_hbm, xu_hbm, xv_hbm,
        idx_u, idx_m, rows_u, rows_m, sem):
    wid = lax.axis_index("s") * NC + lax.axis_index("c")
    base = wid * BPW
    pltpu.sync_copy(user_hbm.at[pl.ds(wid * NCH, NCH)], idx_u)
    pltpu.sync_copy(movie_hbm.at[pl.ds(wid * NCH, NCH)], idx_m)
    copies = []
    for c in range(NCH):
      copies.append(pltpu.async_copy(
          ut_hbm.at[idx_u.at[c]], rows_u.at[pl.ds(c * CH, CH)], sem))
      copies.append(pltpu.async_copy(
          mt_hbm.at[idx_m.at[c]], rows_m.at[pl.ds(c * CH, CH)], sem))
    for cp in copies:
      cp.wait()
    pltpu.sync_copy(rows_u, xu_hbm.at[pl.ds(base, BPW)])
    pltpu.sync_copy(rows_m, xv_hbm.at[pl.ds(base, BPW)])

  return k(user2d, movie2d, user_table, movie_table)


BLK = 2048


def _mlp_body(xu_ref, xv_ref, w1a_ref, w1b_ref, b1_ref, w2_ref, b2_ref,
              out_ref):
  h = jnp.dot(xu_ref[...], w1a_ref[...], preferred_element_type=jnp.float32)
  h = h + jnp.dot(xv_ref[...], w1b_ref[...],
                  preferred_element_type=jnp.float32)
  h = jnp.maximum(h + b1_ref[...], 0.0)
  out_ref[...] = jnp.sum(h * w2_ref[...], axis=1) + b2_ref[0, 0]


def _mlp(xu, xv, w1a, w1b, b1r, w2r, b2r):
  return pl.pallas_call(
      _mlp_body,
      grid=(B // BLK,),
      in_specs=[
          pl.BlockSpec((BLK, D), lambda i: (i, 0)),
          pl.BlockSpec((BLK, D), lambda i: (i, 0)),
          pl.BlockSpec((D, H), lambda i: (0, 0)),
          pl.BlockSpec((D, H), lambda i: (0, 0)),
          pl.BlockSpec((1, H), lambda i: (0, 0)),
          pl.BlockSpec((1, H), lambda i: (0, 0)),
          pl.BlockSpec((1, 1), lambda i: (0, 0)),
      ],
      out_specs=pl.BlockSpec((BLK,), lambda i: (i,)),
      out_shape=jax.ShapeDtypeStruct((B,), jnp.float32),
  )(xu, xv, w1a, w1b, b1r, w2r, b2r)


def kernel(user, movie, user_table, movie_table, W1, b1, W2, b2):
  user2d = user.reshape(B // CH, CH)
  movie2d = movie.reshape(B // CH, CH)
  xu, xv = _sc_gather(user2d, movie2d, user_table, movie_table)
  w1a = W1[:D]
  w1b = W1[D:]
  b1r = b1.reshape(1, H)
  w2r = W2.reshape(1, H)
  b2r = b2.reshape(1, 1)
  return _mlp(xu, xv, w1a, w1b, b1r, w2r, b2r)


# speedup vs baseline: 1.0024x; 1.0024x over previous
import functools

import jax
import jax.numpy as jnp
from jax import lax
from jax.experimental import pallas as pl
from jax.experimental.pallas import tpu as pltpu
from jax.experimental.pallas import tpu_sc as plsc

B = 16384
D = 64
H = 128

NC = 2
NS = 16
NW = NC * NS
BPW = B // NW
CH = 128
NCH = BPW // CH


def _sc_gather(u2, m2, ut2, mt2):
  mesh = plsc.VectorSubcoreMesh(core_axis_name="c", subcore_axis_name="s")

  @functools.partial(
      pl.kernel,
      mesh=mesh,
      out_type=[
          jax.ShapeDtypeStruct((B, 2 * D), jnp.float32),
          jax.ShapeDtypeStruct((B, 2 * D), jnp.float32),
      ],
      scratch_types=[
          pltpu.VMEM((BPW,), jnp.int32),
          pltpu.VMEM((BPW,), jnp.int32),
          pltpu.VMEM((2, CH, 2 * D), jnp.float32),
          pltpu.VMEM((2, CH, 2 * D), jnp.float32),
          pltpu.SemaphoreType.DMA,
      ],
  )
  def k(u_hbm, m_hbm, ut_hbm, mt_hbm, xu_hbm, xv_hbm,
        idx_u, idx_m, rows_u, rows_m, sem):
    wid = lax.axis_index("s") * NC + lax.axis_index("c")
    base = wid * BPW
    pltpu.sync_copy(u_hbm.at[pl.ds(base, BPW)], idx_u)
    pltpu.sync_copy(m_hbm.at[pl.ds(base, BPW)], idx_m)
    inflight = {
        0: (pltpu.async_copy(ut_hbm.at[idx_u.at[pl.ds(0, CH)]],
                             rows_u.at[0], sem),
            pltpu.async_copy(mt_hbm.at[idx_m.at[pl.ds(0, CH)]],
                             rows_m.at[0], sem)),
    }
    for c in range(NCH):
      cu, cm = inflight.pop(c)
      cu.wait()
      cm.wait()
      if c + 1 < NCH:
        s = pl.ds((c + 1) * CH, CH)
        inflight[c + 1] = (
            pltpu.async_copy(ut_hbm.at[idx_u.at[s]],
                             rows_u.at[(c + 1) % 2], sem),
            pltpu.async_copy(mt_hbm.at[idx_m.at[s]],
                             rows_m.at[(c + 1) % 2], sem))
      pltpu.sync_copy(rows_u.at[c % 2], xu_hbm.at[pl.ds(base + c * CH, CH)])
      pltpu.sync_copy(rows_m.at[c % 2], xv_hbm.at[pl.ds(base + c * CH, CH)])

  return k(u2, m2, ut2, mt2)


BLK = 2048


def _mlp_body(xu_ref, xv_ref, su_ref, sv_ref, w1a_ref, w1b_ref, b1_ref,
              w2_ref, b2_ref, out_ref):
  su = su_ref[...]
  sv = sv_ref[...]
  xu = xu_ref[:, :D] + su * (xu_ref[:, D:] - xu_ref[:, :D])
  xv = xv_ref[:, :D] + sv * (xv_ref[:, D:] - xv_ref[:, :D])
  h = jnp.dot(xu, w1a_ref[...], preferred_element_type=jnp.float32)
  h = h + jnp.dot(xv, w1b_ref[...], preferred_element_type=jnp.float32)
  h = jnp.maximum(h + b1_ref[...], 0.0)
  out_ref[...] = jnp.sum(h * w2_ref[...], axis=1) + b2_ref[0, 0]


def _mlp(xu, xv, su, sv, w1a, w1b, b1r, w2r, b2r):
  return pl.pallas_call(
      _mlp_body,
      grid=(B // BLK,),
      in_specs=[
          pl.BlockSpec((BLK, 2 * D), lambda i: (i, 0)),
          pl.BlockSpec((BLK, 2 * D), lambda i: (i, 0)),
          pl.BlockSpec((BLK, 1), lambda i: (i, 0)),
          pl.BlockSpec((BLK, 1), lambda i: (i, 0)),
          pl.BlockSpec((D, H), lambda i: (0, 0)),
          pl.BlockSpec((D, H), lambda i: (0, 0)),
          pl.BlockSpec((1, H), lambda i: (0, 0)),
          pl.BlockSpec((1, H), lambda i: (0, 0)),
          pl.BlockSpec((1, 1), lambda i: (0, 0)),
      ],
      out_specs=pl.BlockSpec((BLK,), lambda i: (i,)),
      out_shape=jax.ShapeDtypeStruct((B,), jnp.float32),
  )(xu, xv, su, sv, w1a, w1b, b1r, w2r, b2r)


def kernel(user, movie, user_table, movie_table, W1, b1, W2, b2):
  ut2 = user_table.reshape(user_table.shape[0] // 2, 2 * D)
  mt2 = movie_table.reshape(movie_table.shape[0] // 2, 2 * D)
  u2 = lax.shift_right_logical(user, 1)
  m2 = lax.shift_right_logical(movie, 1)
  su = (user & 1).astype(jnp.float32).reshape(B, 1)
  sv = (movie & 1).astype(jnp.float32).reshape(B, 1)
  xu, xv = _sc_gather(u2, m2, ut2, mt2)
  w1a = W1[:D]
  w1b = W1[D:]
  b1r = b1.reshape(1, H)
  w2r = W2.reshape(1, H)
  b2r = b2.reshape(1, 1)
  return _mlp(xu, xv, su, sv, w1a, w1b, b1r, w2r, b2r)


# speedup vs baseline: 1.4293x; 1.4259x over previous
import functools

import jax
import jax.numpy as jnp
from jax import lax
from jax.experimental import pallas as pl
from jax.experimental.pallas import tpu as pltpu
from jax.experimental.pallas import tpu_sc as plsc

B = 16384
D = 64
H = 128
SL = 8

NC = 2
NS = 16
NW = NC * NS
BPW = B // NW
K = 16
NG = BPW // K


def _sc_gather(user, movie, ut, mt):
  mesh = plsc.VectorSubcoreMesh(core_axis_name="c", subcore_axis_name="s")

  @functools.partial(
      pl.kernel,
      mesh=mesh,
      out_type=[
          jax.ShapeDtypeStruct((B, D), jnp.float32),
          jax.ShapeDtypeStruct((B, D), jnp.float32),
      ],
      scratch_types=[
          pltpu.VMEM((BPW,), jnp.int32),
          pltpu.VMEM((BPW,), jnp.int32),
          pltpu.VMEM((K, SL, D), jnp.float32),
          pltpu.VMEM((K, SL, D), jnp.float32),
          pltpu.VMEM((K, D), jnp.float32),
          pltpu.VMEM((K, D), jnp.float32),
          pltpu.SemaphoreType.DMA,
          pltpu.SemaphoreType.DMA,
      ],
  )
  def k(ur_hbm, mr_hbm, ut_hbm, mt_hbm, xu_hbm, xv_hbm,
        raw_u_v, raw_m_v,
        ring_u, ring_m, rows_u, rows_m, sem_u, sem_m):
    wid = lax.axis_index("s") * NC + lax.axis_index("c")
    base = wid * BPW
    pltpu.sync_copy(ur_hbm.at[pl.ds(base, BPW)], raw_u_v)
    pltpu.sync_copy(mr_hbm.at[pl.ds(base, BPW)], raw_m_v)

    @pl.loop(0, NG)
    def grp(g):
      vu = raw_u_v[pl.ds(g * K, K)]
      vm = raw_m_v[pl.ds(g * K, K)]
      waits = []
      for j in range(K):
        su = pl.multiple_of((vu[j] >> 3) * SL, SL)
        waits.append(pltpu.async_copy(
            ut_hbm.at[pl.ds(su, SL)], ring_u.at[j], sem_u))
        sm = pl.multiple_of((vm[j] >> 3) * SL, SL)
        waits.append(pltpu.async_copy(
            mt_hbm.at[pl.ds(sm, SL)], ring_m.at[j], sem_m))
      for cp in waits:
        cp.wait()
      for j in range(K):
        sub_u = vu[j] & (SL - 1)
        sub_m = vm[j] & (SL - 1)
        for kk in range(D // 16):
          c = pl.ds(kk * 16, 16)
          rows_u[j, c] = ring_u[j, sub_u, c]
          rows_m[j, c] = ring_m[j, sub_m, c]
      pltpu.sync_copy(rows_u, xu_hbm.at[pl.ds(base + g * K, K)])
      pltpu.sync_copy(rows_m, xv_hbm.at[pl.ds(base + g * K, K)])

  return k(user, movie, ut, mt)


BLK = 2048


def _mlp_body(xu_ref, xv_ref, w1a_ref, w1b_ref, b1_ref, w2_ref, b2_ref,
              out_ref):
  h = jnp.dot(xu_ref[...], w1a_ref[...], preferred_element_type=jnp.float32)
  h = h + jnp.dot(xv_ref[...], w1b_ref[...],
                  preferred_element_type=jnp.float32)
  h = jnp.maximum(h + b1_ref[...], 0.0)
  out_ref[...] = jnp.sum(h * w2_ref[...], axis=1) + b2_ref[0, 0]


def _mlp(xu, xv, w1a, w1b, b1r, w2r, b2r):
  return pl.pallas_call(
      _mlp_body,
      grid=(B // BLK,),
      in_specs=[
          pl.BlockSpec((BLK, D), lambda i: (i, 0)),
          pl.BlockSpec((BLK, D), lambda i: (i, 0)),
          pl.BlockSpec((D, H), lambda i: (0, 0)),
          pl.BlockSpec((D, H), lambda i: (0, 0)),
          pl.BlockSpec((1, H), lambda i: (0, 0)),
          pl.BlockSpec((1, H), lambda i: (0, 0)),
          pl.BlockSpec((1, 1), lambda i: (0, 0)),
      ],
      out_specs=pl.BlockSpec((BLK,), lambda i: (i,)),
      out_shape=jax.ShapeDtypeStruct((B,), jnp.float32),
  )(xu, xv, w1a, w1b, b1r, w2r, b2r)


def kernel(user, movie, user_table, movie_table, W1, b1, W2, b2):
  xu, xv = _sc_gather(user, movie, user_table, movie_table)
  w1a = W1[:D]
  w1b = W1[D:]
  b1r = b1.reshape(1, H)
  w2r = W2.reshape(1, H)
  b2r = b2.reshape(1, 1)
  return _mlp(xu, xv, w1a, w1b, b1r, w2r, b2r)
